# Initial kernel scaffold; baseline (speedup 1.0000x reference)
import jax
import jax.numpy as jnp
from jax.experimental import pallas as pl


def kernel(edge_attr, n_id, e_id, edge_index, entity_emb, relation_emb, relation_matrix):
    raise NotImplementedError("write your pallas kernel here")


# speedup vs baseline: 6.0758x; 6.0758x over previous
import functools

import jax
import jax.numpy as jnp
from jax import lax
from jax.experimental import pallas as pl
from jax.experimental.pallas import tpu as pltpu
from jax.experimental.pallas import tpu_sc as plsc

N_ENT = 10000
N_REL = 8
N_EDGE = 160000
D = 128

NC, NS, L = 2, 16, 16
NW = NC * NS
CH = 128
NCHUNK = N_EDGE // CH
ROWS_X = N_ENT // 25

_mesh = plsc.VectorSubcoreMesh(core_axis_name="c", subcore_axis_name="s")


@functools.partial(
    pl.kernel,
    mesh=_mesh,
    out_type=[
        jax.ShapeDtypeStruct((N_ENT, D), jnp.float32),
        jax.ShapeDtypeStruct((N_EDGE,), jnp.int32),
    ],
    scratch_types=[
        pltpu.VMEM((CH,), jnp.int32),
        pltpu.VMEM((CH,), jnp.int32),
        pltpu.VMEM((CH,), jnp.int32),
        pltpu.VMEM((CH,), jnp.int32),
        pltpu.VMEM((CH,), jnp.int32),
        pltpu.VMEM((ROWS_X,), jnp.int32),
        pltpu.VMEM((80, D), jnp.float32),
        pltpu.SemaphoreType.DMA,
    ],
)
def _prep(nid_hbm, eid_hbm, eaflat_hbm, src_hbm, emb_hbm,
          x_hbm, gidx_hbm,
          eidv, fidv, typev, srcv, gidxv, nidv, rows80, sem):
    c = lax.axis_index("c")
    s = lax.axis_index("s")
    wid = s * NC + c

    nch = jnp.where(wid < (NCHUNK - (NCHUNK // NW) * NW),
                    NCHUNK // NW + 1, NCHUNK // NW)

    def chunk_body(i, _):
        ch = wid + i * NW
        base = ch * CH
        pltpu.sync_copy(eid_hbm.at[pl.ds(base, CH)], eidv)
        for j in range(CH // L):
            fidv[pl.ds(j * L, L)] = eidv[pl.ds(j * L, L)] * 2 + 1
        pltpu.async_copy(eaflat_hbm.at[fidv], typev, sem).wait()
        pltpu.sync_copy(src_hbm.at[pl.ds(base, CH)], srcv)
        for j in range(CH // L):
            gidxv[pl.ds(j * L, L)] = (
                typev[pl.ds(j * L, L)] * N_ENT + srcv[pl.ds(j * L, L)])
        pltpu.sync_copy(gidxv, gidx_hbm.at[pl.ds(base, CH)])
        return 0

    lax.fori_loop(0, nch, chunk_body, 0)

    @pl.when(wid < N_ENT // ROWS_X)
    def _():
        base = wid * ROWS_X
        pltpu.sync_copy(nid_hbm.at[pl.ds(base, ROWS_X)], nidv)
        for j in range(ROWS_X // 80):
            pltpu.async_copy(
                emb_hbm.at[nidv.at[pl.ds(j * 80, 80)]], rows80, sem).wait()
            pltpu.sync_copy(rows80, x_hbm.at[pl.ds(base + j * 80, 80)])


def _ymat_body(x_ref, r_ref, y_ref):
    y_ref[0] = jnp.dot(x_ref[...], r_ref[0], preferred_element_type=jnp.float32)


def _ymat(x, relation_matrix):
    nb = N_ENT // ROWS_X
    return pl.pallas_call(
        _ymat_body,
        grid=(N_REL + 1, nb),
        in_specs=[
            pl.BlockSpec((ROWS_X, D), lambda r, j: (j, 0)),
            pl.BlockSpec((1, D, D), lambda r, j: (r, 0, 0)),
        ],
        out_specs=pl.BlockSpec((1, ROWS_X, D), lambda r, j: (r, j, 0)),
        out_shape=jax.ShapeDtypeStruct((N_REL + 1, N_ENT, D), jnp.float32),
    )(x, relation_matrix)


ACC_PAD = 10240
ROWS_T = ACC_PAD // NS
CNT_PAD = 10240


@functools.partial(
    pl.kernel,
    mesh=_mesh,
    out_type=[
        jax.ShapeDtypeStruct((NC, ACC_PAD, D), jnp.float32),
        jax.ShapeDtypeStruct((NC, N_ENT), jnp.float32),
    ],
    scratch_types=[
        pltpu.VMEM((CH,), jnp.int32),
        pltpu.VMEM((CH,), jnp.int32),
        pltpu.VMEM((CH, D), jnp.float32),
        pltpu.VMEM((CH,), jnp.float32),
        pltpu.VMEM((CH, D), jnp.float32),
        pltpu.VMEM((CNT_PAD // NS,), jnp.float32),
        pltpu.VMEM((N_ENT,), jnp.float32),
        pltpu.VMEM_SHARED((ACC_PAD, D), jnp.float32),
        pltpu.VMEM_SHARED((CNT_PAD,), jnp.float32),
        pltpu.SemaphoreType.DMA,
    ],
)
def _scatter(gidx_hbm, dst_hbm, yflat_hbm,
             pacc_hbm, pcnt_hbm,
             gidxv, dstv, rows, ones, zbuf, zflat, cbuf, accS, cntS, sem):
    c = lax.axis_index("c")
    s = lax.axis_index("s")
    wid = s * NC + c

    def zb_body(i, _):
        for j in range(D // L):
            zbuf[i, pl.ds(j * L, L)] = jnp.zeros((L,), jnp.float32)
        return 0

    lax.fori_loop(0, CH, zb_body, 0)

    def zf_body(k, _):
        zflat[pl.ds(k * L, L)] = jnp.zeros((L,), jnp.float32)
        return 0

    lax.fori_loop(0, (CNT_PAD // NS) // L, zf_body, 0)
    for j in range(CH // L):
        ones[pl.ds(j * L, L)] = jnp.ones((L,), jnp.float32)

    for k in range(ROWS_T // CH):
        pltpu.sync_copy(zbuf, accS.at[pl.ds(s * ROWS_T + k * CH, CH)])
    pltpu.sync_copy(zflat, cntS.at[pl.ds(s * (CNT_PAD // NS), CNT_PAD // NS)])
    plsc.subcore_barrier()

    nch = jnp.where(wid < (NCHUNK - (NCHUNK // NW) * NW),
                    NCHUNK // NW + 1, NCHUNK // NW)

    def chunk_body(i, _):
        ch = wid + i * NW
        base = ch * CH
        pltpu.sync_copy(gidx_hbm.at[pl.ds(base, CH)], gidxv)
        pltpu.async_copy(yflat_hbm.at[gidxv], rows, sem).wait()
        pltpu.sync_copy(dst_hbm.at[pl.ds(base, CH)], dstv)
        pltpu.sync_copy(rows, accS.at[dstv], add=True)
        pltpu.sync_copy(ones, cntS.at[dstv], add=True)
        return 0

    lax.fori_loop(0, nch, chunk_body, 0)
    plsc.subcore_barrier()

    for k in range(ROWS_T // CH):
        b = s * ROWS_T + k * CH
        pltpu.sync_copy(accS.at[pl.ds(b, CH)], zbuf)
        pltpu.sync_copy(zbuf, pacc_hbm.at[c, pl.ds(b, CH)])

    @pl.when(s == 0)
    def _():
        pltpu.sync_copy(cntS.at[pl.ds(0, N_ENT)], cbuf)
        pltpu.sync_copy(cbuf, pcnt_hbm.at[c])


def _fin_body(p_ref, c_ref, ys_ref, o_ref):
    tot = 1.0 + c_ref[0] + c_ref[1]
    o_ref[...] = (p_ref[0] + p_ref[1] + ys_ref[...]) / tot


def _finish(pacc, pcnt, yflat):
    nb = N_ENT // ROWS_X
    yself_row0 = (N_REL * N_ENT) // ROWS_X
    return pl.pallas_call(
        _fin_body,
        grid=(nb,),
        in_specs=[
            pl.BlockSpec((NC, ROWS_X, D), lambda j: (0, j, 0)),
            pl.BlockSpec((NC, ROWS_X, 1), lambda j: (0, j, 0)),
            pl.BlockSpec((ROWS_X, D), lambda j: (yself_row0 + j, 0)),
        ],
        out_specs=pl.BlockSpec((ROWS_X, D), lambda j: (j, 0)),
        out_shape=jax.ShapeDtypeStruct((N_ENT, D), jnp.float32),
    )(pacc, pcnt.reshape(NC, N_ENT, 1), yflat)


def kernel(edge_attr, n_id, e_id, edge_index, entity_emb, relation_emb,
           relation_matrix):
    del relation_emb
    src = edge_index[0]
    dst = edge_index[1]
    ea_flat = edge_attr.reshape(-1)

    x, gidx = _prep(n_id, e_id, ea_flat, src, entity_emb)
    yall = _ymat(x, relation_matrix)
    yflat = yall.reshape((N_REL + 1) * N_ENT, D)
    pacc, pcnt = _scatter(gidx, dst, yflat)
    out = _finish(pacc, pcnt, yflat)
    return out, n_id, e_id, edge_index


# speedup vs baseline: 7.2504x; 1.1933x over previous
import functools

import jax
import jax.numpy as jnp
from jax import lax
from jax.experimental import pallas as pl
from jax.experimental.pallas import tpu as pltpu
from jax.experimental.pallas import tpu_sc as plsc

N_ENT = 10000
N_REL = 8
N_EDGE = 160000
D = 128

NC, NS, L = 2, 16, 16
NW = NC * NS
CH = 128
NCHUNK = N_EDGE // CH
ROWS_X = N_ENT // 25

_mesh = plsc.VectorSubcoreMesh(core_axis_name="c", subcore_axis_name="s")


@functools.partial(
    pl.kernel,
    mesh=_mesh,
    out_type=jax.ShapeDtypeStruct((N_ENT, D), jnp.float32),
    scratch_types=[
        pltpu.VMEM((ROWS_X,), jnp.int32),
        pltpu.VMEM((ROWS_X, D), jnp.float32),
        pltpu.SemaphoreType.DMA,
    ],
)
def _prep(nid_hbm, emb_hbm, x_hbm, nidv, rowsx, sem):
    c = lax.axis_index("c")
    s = lax.axis_index("s")
    wid = s * NC + c

    @pl.when(wid < N_ENT // ROWS_X)
    def _():
        base = wid * ROWS_X
        pltpu.sync_copy(nid_hbm.at[pl.ds(base, ROWS_X)], nidv)
        cps = [
            pltpu.async_copy(
                emb_hbm.at[nidv.at[pl.ds(j * 80, 80)]],
                rowsx.at[pl.ds(j * 80, 80)], sem)
            for j in range(ROWS_X // 80)
        ]
        for cp in cps:
            cp.wait()
        pltpu.sync_copy(rowsx, x_hbm.at[pl.ds(base, ROWS_X)])


def _ymat_body(x_ref, r_ref, y_ref):
    y_ref[0] = jnp.dot(x_ref[...], r_ref[0], preferred_element_type=jnp.float32)


def _ymat(x, relation_matrix):
    nb = N_ENT // ROWS_X
    return pl.pallas_call(
        _ymat_body,
        grid=(N_REL + 1, nb),
        in_specs=[
            pl.BlockSpec((ROWS_X, D), lambda r, j: (j, 0)),
            pl.BlockSpec((1, D, D), lambda r, j: (r, 0, 0)),
        ],
        out_specs=pl.BlockSpec((1, ROWS_X, D), lambda r, j: (r, j, 0)),
        out_shape=jax.ShapeDtypeStruct((N_REL + 1, N_ENT, D), jnp.float32),
    )(x, relation_matrix)


ACC_PAD = 10240
ROWS_T = ACC_PAD // NS
CNT_PAD = 10240


@functools.partial(
    pl.kernel,
    mesh=_mesh,
    out_type=[
        jax.ShapeDtypeStruct((NC, ACC_PAD, D), jnp.float32),
        jax.ShapeDtypeStruct((NC, CNT_PAD), jnp.float32),
    ],
    scratch_types=[
        pltpu.VMEM((CH,), jnp.int32),
        pltpu.VMEM((CH,), jnp.int32),
        pltpu.VMEM((CH,), jnp.int32),
        pltpu.VMEM((CH,), jnp.int32),
        pltpu.VMEM((2, CH), jnp.int32),
        pltpu.VMEM((2, CH), jnp.int32),
        pltpu.VMEM((2, CH, D), jnp.float32),
        pltpu.VMEM((CH,), jnp.float32),
        pltpu.VMEM((CNT_PAD // NS,), jnp.float32),
        pltpu.VMEM_SHARED((ACC_PAD, D), jnp.float32),
        pltpu.VMEM_SHARED((CNT_PAD,), jnp.float32),
        pltpu.SemaphoreType.DMA,
        pltpu.SemaphoreType.DMA,
        pltpu.SemaphoreType.DMA,
    ],
)
def _scatter(eid_hbm, eaflat_hbm, src_hbm, dst_hbm, yflat_hbm,
             pacc_hbm, pcnt_hbm,
             eidv, fidv, typev, srcv, gidx2, dst2, rows2, ones, zflat,
             accS, cntS, semT, semG0, semG1):
    c = lax.axis_index("c")
    s = lax.axis_index("s")
    wid = s * NC + c
    semG = (semG0, semG1)

    def zb_body(i, _):
        for j in range(D // L):
            rows2[0, i, pl.ds(j * L, L)] = jnp.zeros((L,), jnp.float32)
        return 0

    lax.fori_loop(0, CH, zb_body, 0)

    def zf_body(k, _):
        zflat[pl.ds(k * L, L)] = jnp.zeros((L,), jnp.float32)
        return 0

    lax.fori_loop(0, (CNT_PAD // NS) // L, zf_body, 0)
    for j in range(CH // L):
        ones[pl.ds(j * L, L)] = jnp.ones((L,), jnp.float32)

    for k in range(ROWS_T // CH):
        pltpu.sync_copy(rows2.at[0], accS.at[pl.ds(s * ROWS_T + k * CH, CH)])
    pltpu.sync_copy(zflat, cntS.at[pl.ds(s * (CNT_PAD // NS), CNT_PAD // NS)])
    plsc.subcore_barrier()

    nch = jnp.where(wid < (NCHUNK - (NCHUNK // NW) * NW),
                    NCHUNK // NW + 1, NCHUNK // NW)

    def prep_chunk(i, p):
        base = (wid + i * NW) * CH
        pltpu.sync_copy(eid_hbm.at[pl.ds(base, CH)], eidv)
        for j in range(CH // L):
            fidv[pl.ds(j * L, L)] = eidv[pl.ds(j * L, L)] * 2 + 1
        pltpu.async_copy(eaflat_hbm.at[fidv], typev, semT).wait()
        pltpu.sync_copy(src_hbm.at[pl.ds(base, CH)], srcv)
        for j in range(CH // L):
            gidx2[p, pl.ds(j * L, L)] = (
                typev[pl.ds(j * L, L)] * N_ENT + srcv[pl.ds(j * L, L)])
        pltpu.sync_copy(dst_hbm.at[pl.ds(base, CH)], dst2.at[p])
        pltpu.async_copy(yflat_hbm.at[gidx2.at[p]], rows2.at[p], semG[p])

    def cons_chunk(p):
        pltpu.make_async_copy(
            yflat_hbm.at[gidx2.at[p]], rows2.at[p], semG[p]).wait()
        pltpu.sync_copy(rows2.at[p], accS.at[dst2.at[p]], add=True)
        pltpu.sync_copy(ones, cntS.at[dst2.at[p]], add=True)

    prep_chunk(jnp.int32(0), 0)

    def pair_body(q, _):
        i0 = q * 2

        @pl.when(i0 + 1 < nch)
        def _():
            prep_chunk(i0 + 1, 1)

        cons_chunk(0)

        @pl.when(i0 + 2 < nch)
        def _():
            prep_chunk(i0 + 2, 0)

        @pl.when(i0 + 1 < nch)
        def _():
            cons_chunk(1)

        return 0

    lax.fori_loop(0, (nch + 1) // 2, pair_body, 0)
    plsc.subcore_barrier()

    for k in range(ROWS_T // CH):
        b = s * ROWS_T + k * CH
        pltpu.sync_copy(accS.at[pl.ds(b, CH)], rows2.at[0])
        pltpu.sync_copy(rows2.at[0], pacc_hbm.at[c, pl.ds(b, CH)])

    cb = s * (CNT_PAD // NS)
    pltpu.sync_copy(cntS.at[pl.ds(cb, CNT_PAD // NS)], zflat)
    pltpu.sync_copy(zflat, pcnt_hbm.at[c, pl.ds(cb, CNT_PAD // NS)])


def _fin_body(p_ref, c_ref, ys_ref, o_ref):
    tot = 1.0 + c_ref[0] + c_ref[1]
    o_ref[...] = (p_ref[0] + p_ref[1] + ys_ref[...]) / tot


def _finish(pacc, pcnt, yflat):
    nb = N_ENT // ROWS_X
    yself_row0 = (N_REL * N_ENT) // ROWS_X
    return pl.pallas_call(
        _fin_body,
        grid=(nb,),
        in_specs=[
            pl.BlockSpec((NC, ROWS_X, D), lambda j: (0, j, 0)),
            pl.BlockSpec((NC, ROWS_X, 1), lambda j: (0, j, 0)),
            pl.BlockSpec((ROWS_X, D), lambda j: (yself_row0 + j, 0)),
        ],
        out_specs=pl.BlockSpec((ROWS_X, D), lambda j: (j, 0)),
        out_shape=jax.ShapeDtypeStruct((N_ENT, D), jnp.float32),
    )(pacc, pcnt.reshape(NC, CNT_PAD, 1), yflat)


def kernel(edge_attr, n_id, e_id, edge_index, entity_emb, relation_emb,
           relation_matrix):
    del relation_emb
    src = edge_index[0]
    dst = edge_index[1]
    ea_flat = edge_attr.reshape(-1)

    x = _prep(n_id, entity_emb)
    yall = _ymat(x, relation_matrix)
    yflat = yall.reshape((N_REL + 1) * N_ENT, D)
    pacc, pcnt = _scatter(e_id, ea_flat, src, dst, yflat)
    out = _finish(pacc, pcnt, yflat)
    return out, n_id, e_id, edge_index
